# Initial kernel scaffold; baseline (speedup 1.0000x reference)
import jax
import jax.numpy as jnp
from jax.experimental import pallas as pl


def kernel(grad, sort_indices, pos_idx):
    raise NotImplementedError("write your pallas kernel here")


# speedup vs baseline: 2.7735x; 2.7735x over previous
import functools

import jax
import jax.numpy as jnp
from jax import lax
from jax.experimental import pallas as pl
from jax.experimental.pallas import tpu as pltpu
from jax.experimental.pallas import tpu_sc as plsc

_NUM_ROWS_OUT = 100000
_D = 64
_N = 4096 * 200
_B = 256
_NW = 32
_P = 2
_NCHUNK = _NW * _P
_CH = 1568
_VPAD = _NCHUNK * _CH
_NLANE = 16


def _make_sc_kernel():
  mesh = plsc.VectorSubcoreMesh(core_axis_name="c", subcore_axis_name="s")

  @functools.partial(
      pl.kernel,
      mesh=mesh,
      compiler_params=pltpu.CompilerParams(needs_layout_passes=False),
      out_type=jax.ShapeDtypeStruct((_VPAD * _D,), jnp.float32),
      scratch_types=[
          pltpu.VMEM((128,), jnp.int32),
          pltpu.VMEM((_B,), jnp.int32),
          pltpu.VMEM((_B * _D,), jnp.float32),
          pltpu.VMEM((_CH * _D,), jnp.float32),
      ],
  )
  def scatter_add_kernel(grad_hbm, idx_hbm, starts_hbm, zeros_hbm, out_hbm,
                         starts_v, idxblk_v, stage_v, accum_v):
    wid = lax.axis_index("s") * 2 + lax.axis_index("c")
    pltpu.sync_copy(starts_hbm, starts_v)
    lane = lax.iota(jnp.int32, _NLANE)
    cols = [lane + 16 * j for j in range(_D // _NLANE)]

    for p in range(_P):
      c = wid * _P + p
      lo = c * _CH
      pltpu.sync_copy(zeros_hbm, accum_v)
      bounds_vec = starts_v[pl.ds(c, _NLANE)]
      row_start = bounds_vec[0]
      row_end = bounds_vec[1]
      blk_lo = row_start // _B
      blk_hi = (row_end + _B - 1) // _B

      def blk_body(k, carry):
        pltpu.sync_copy(grad_hbm.at[pl.ds(k * (_B * _D), _B * _D)], stage_v)
        pltpu.sync_copy(idx_hbm.at[pl.ds(k * _B, _B)], idxblk_v)

        def grp_body(g, rcarry):
          locv16 = idxblk_v[pl.ds(g * _NLANE, _NLANE)] - lo
          for t in range(_NLANE):
            locb = jnp.full((_NLANE,), locv16[t], dtype=jnp.int32)
            mask = (locb >= 0) & (locb < _CH)
            base = locb * _D
            b = g * _NLANE + t
            for j in range(_D // _NLANE):
              v = stage_v[pl.ds(b * _D + 16 * j, _NLANE)]
              plsc.addupdate_scatter(accum_v, [base + cols[j]], v, mask=mask)
          return rcarry

        lax.fori_loop(0, _B // _NLANE, grp_body, 0)
        return carry

      lax.fori_loop(blk_lo, blk_hi, blk_body, 0)

      @pl.when(c == 0)
      def _zero_padding_row():
        z = jnp.zeros((_NLANE,), jnp.float32)
        for j in range(_D // _NLANE):
          accum_v[pl.ds(16 * j, _NLANE)] = z

      pltpu.sync_copy(accum_v, out_hbm.at[pl.ds(lo * _D, _CH * _D)])

  return scatter_add_kernel


_SC_KERNEL = _make_sc_kernel()


@jax.jit
def kernel(grad, sort_indices, pos_idx):
  del pos_idx
  g = grad.reshape(-1)
  idx = sort_indices.reshape(-1).astype(jnp.int32)
  bounds = jnp.arange(_NCHUNK + 1, dtype=jnp.int32) * _CH
  starts = jnp.searchsorted(idx, bounds, side="left").astype(jnp.int32)
  starts_padded = jnp.zeros((128,), jnp.int32).at[: _NCHUNK + 1].set(starts)
  zeros = jnp.zeros((_CH * _D,), jnp.float32)
  out = _SC_KERNEL(g, idx, starts_padded, zeros)
  return out.reshape(_VPAD, _D)[:_NUM_ROWS_OUT]


# speedup vs baseline: 3.0303x; 1.0926x over previous
import functools

import jax
import jax.numpy as jnp
from jax import lax
from jax.experimental import pallas as pl
from jax.experimental.pallas import tpu as pltpu
from jax.experimental.pallas import tpu_sc as plsc

_NUM_ROWS_OUT = 100000
_D = 64
_N = 4096 * 200
_B = 256
_NW = 32
_P = 4
_NCHUNK = _NW * _P
_CH = 784
_VPAD = _NCHUNK * _CH
_NLANE = 16


def _make_sc_kernel():
  mesh = plsc.VectorSubcoreMesh(core_axis_name="c", subcore_axis_name="s")

  @functools.partial(
      pl.kernel,
      mesh=mesh,
      compiler_params=pltpu.CompilerParams(needs_layout_passes=False),
      out_type=jax.ShapeDtypeStruct((_VPAD * _D,), jnp.float32),
      scratch_types=[
          pltpu.VMEM((256,), jnp.int32),
          pltpu.VMEM((2, _B), jnp.int32),
          pltpu.VMEM((2, _B * _D), jnp.float32),
          pltpu.VMEM((_CH * _D,), jnp.float32),
          pltpu.SemaphoreType.DMA,
          pltpu.SemaphoreType.DMA,
          pltpu.SemaphoreType.DMA,
          pltpu.SemaphoreType.DMA,
      ],
  )
  def scatter_add_kernel(grad_hbm, idx_hbm, starts_hbm, zeros_hbm, out_hbm,
                         starts_v, idxblk_v, stage_v, accum_v,
                         gsem0, gsem1, isem0, isem1):
    wid = lax.axis_index("s") * 2 + lax.axis_index("c")
    pltpu.sync_copy(starts_hbm, starts_v)
    lane = lax.iota(jnp.int32, _NLANE)
    cols = [lane + 16 * j for j in range(_D // _NLANE)]
    gsems = (gsem0, gsem1)
    isems = (isem0, isem1)

    def start_fetch(k, buf):
      pltpu.make_async_copy(
          grad_hbm.at[pl.ds(k * (_B * _D), _B * _D)],
          stage_v.at[buf], gsems[buf]).start()
      pltpu.make_async_copy(
          idx_hbm.at[pl.ds(k * _B, _B)],
          idxblk_v.at[buf], isems[buf]).start()

    def wait_fetch(buf):
      pltpu.make_async_copy(
          grad_hbm.at[pl.ds(0, _B * _D)],
          stage_v.at[buf], gsems[buf]).wait()
      pltpu.make_async_copy(
          idx_hbm.at[pl.ds(0, _B)],
          idxblk_v.at[buf], isems[buf]).wait()

    for p in range(_P):
      c = wid * _P + p
      lo = c * _CH
      pltpu.sync_copy(zeros_hbm, accum_v)
      bounds_vec = starts_v[pl.ds(c, _NLANE)]
      row_start = bounds_vec[0]
      row_end = bounds_vec[1]
      blk_lo = row_start // _B
      blk_hi = (row_end + _B - 1) // _B

      @pl.when(blk_lo < blk_hi)
      def _prime0():
        start_fetch(blk_lo, 0)

      @pl.when(blk_lo + 1 < blk_hi)
      def _prime1():
        start_fetch(blk_lo + 1, 1)

      def consume(buf, kb):
        wait_fetch(buf)

        def grp_body(g, rcarry):
          locv16 = idxblk_v[buf, pl.ds(g * _NLANE, _NLANE)] - lo
          for t in range(_NLANE):
            locb = jnp.full((_NLANE,), locv16[t], dtype=jnp.int32)
            mask = (locb >= 0) & (locb < _CH)
            base = locb * _D
            b = g * _NLANE + t
            for j in range(_D // _NLANE):
              v = stage_v[buf, pl.ds(b * _D + 16 * j, _NLANE)]
              plsc.addupdate_scatter(accum_v, [base + cols[j]], v, mask=mask)
          return rcarry

        lax.fori_loop(0, _B // _NLANE, grp_body, 0)

        @pl.when(kb + 2 < blk_hi)
        def _next():
          start_fetch(kb + 2, buf)

      def pair_body(i, carry):
        for buf in range(2):
          kb = blk_lo + 2 * i + buf

          @pl.when(kb < blk_hi)
          def _consume():
            consume(buf, kb)

        return carry

      npairs = (blk_hi - blk_lo + 1) // 2
      lax.fori_loop(0, npairs, pair_body, 0)

      @pl.when(c == 0)
      def _zero_padding_row():
        z = jnp.zeros((_NLANE,), jnp.float32)
        for j in range(_D // _NLANE):
          accum_v[pl.ds(16 * j, _NLANE)] = z

      pltpu.sync_copy(accum_v, out_hbm.at[pl.ds(lo * _D, _CH * _D)])

  return scatter_add_kernel


_SC_KERNEL = _make_sc_kernel()


@jax.jit
def kernel(grad, sort_indices, pos_idx):
  del pos_idx
  g = grad.reshape(-1)
  idx = sort_indices.reshape(-1).astype(jnp.int32)
  bounds = jnp.arange(_NCHUNK + 1, dtype=jnp.int32) * _CH
  starts = jnp.searchsorted(idx, bounds, side="left").astype(jnp.int32)
  starts_padded = jnp.zeros((256,), jnp.int32).at[: _NCHUNK + 1].set(starts)
  zeros = jnp.zeros((_CH * _D,), jnp.float32)
  out = _SC_KERNEL(g, idx, starts_padded, zeros)
  return out.reshape(_VPAD, _D)[:_NUM_ROWS_OUT]


# speedup vs baseline: 3.9714x; 1.3106x over previous
import functools

import jax
import jax.numpy as jnp
from jax import lax
from jax.experimental import pallas as pl
from jax.experimental.pallas import tpu as pltpu
from jax.experimental.pallas import tpu_sc as plsc

_NUM_ROWS_OUT = 100000
_D = 64
_N = 4096 * 200
_B = 256
_NW = 32
_P = 4
_NCHUNK = _NW * _P
_CH = 784
_VPAD = _NCHUNK * _CH
_NLANE = 16


def _make_sc_kernel():
  mesh = plsc.VectorSubcoreMesh(core_axis_name="c", subcore_axis_name="s")

  @functools.partial(
      pl.kernel,
      mesh=mesh,
      compiler_params=pltpu.CompilerParams(needs_layout_passes=False),
      out_type=jax.ShapeDtypeStruct((_VPAD * _D,), jnp.float32),
      scratch_types=[
          pltpu.VMEM((256,), jnp.int32),
          pltpu.VMEM((2, _B), jnp.int32),
          pltpu.VMEM((2, _B * _D), jnp.float32),
          pltpu.VMEM((_CH * _D,), jnp.float32),
          pltpu.SemaphoreType.DMA,
          pltpu.SemaphoreType.DMA,
          pltpu.SemaphoreType.DMA,
          pltpu.SemaphoreType.DMA,
      ],
  )
  def scatter_add_kernel(grad_hbm, idx_hbm, starts_hbm, zeros_hbm, out_hbm,
                         starts_v, idxblk_v, stage_v, accum_v,
                         gsem0, gsem1, isem0, isem1):
    wid = lax.axis_index("s") * 2 + lax.axis_index("c")
    pltpu.sync_copy(starts_hbm, starts_v)
    lane = lax.iota(jnp.int32, _NLANE)
    cols = [lane + 16 * j for j in range(_D // _NLANE)]
    gsems = (gsem0, gsem1)
    isems = (isem0, isem1)

    def start_fetch(k, buf):
      pltpu.make_async_copy(
          grad_hbm.at[pl.ds(k * (_B * _D), _B * _D)],
          stage_v.at[buf], gsems[buf]).start()
      pltpu.make_async_copy(
          idx_hbm.at[pl.ds(k * _B, _B)],
          idxblk_v.at[buf], isems[buf]).start()

    def wait_fetch(buf):
      pltpu.make_async_copy(
          grad_hbm.at[pl.ds(0, _B * _D)],
          stage_v.at[buf], gsems[buf]).wait()
      pltpu.make_async_copy(
          idx_hbm.at[pl.ds(0, _B)],
          idxblk_v.at[buf], isems[buf]).wait()

    for p in range(_P):
      c = wid * _P + p
      lo = c * _CH
      pltpu.sync_copy(zeros_hbm, accum_v)
      bounds_vec = starts_v[pl.ds(c, _NLANE)]
      row_start = bounds_vec[0]
      row_end = bounds_vec[1]
      blk_lo = row_start // _B
      blk_hi = (row_end + _B - 1) // _B

      @pl.when(blk_lo < blk_hi)
      def _prime0():
        start_fetch(blk_lo, 0)

      @pl.when(blk_lo + 1 < blk_hi)
      def _prime1():
        start_fetch(blk_lo + 1, 1)

      def consume(buf, kb):
        wait_fetch(buf)

        def grp_body(g, rcarry):
          locv16 = idxblk_v[buf, pl.ds(g * _NLANE, _NLANE)] - lo
          nj = _D // _NLANE
          cur_locb = jnp.full((_NLANE,), -(2 ** 30), dtype=jnp.int32)
          cur_base = cur_locb
          cur_validm = cur_locb < cur_locb
          acc = [jnp.zeros((_NLANE,), jnp.float32) for _ in range(nj)]
          for t in range(_NLANE):
            locb = jnp.full((_NLANE,), locv16[t], dtype=jnp.int32)
            samev = locb == cur_locb
            flushm = (~samev) & cur_validm
            b = g * _NLANE + t
            v = [stage_v[buf, pl.ds(b * _D + 16 * j, _NLANE)]
                 for j in range(nj)]
            for j in range(nj):
              plsc.addupdate_scatter(accum_v, [cur_base + cols[j]], acc[j],
                                     mask=flushm)
            for j in range(nj):
              acc[j] = jnp.where(samev, acc[j] + v[j], v[j])
            cur_locb = locb
            cur_base = locb * _D
            cur_validm = (locb >= 0) & (locb < _CH)
          for j in range(nj):
            plsc.addupdate_scatter(accum_v, [cur_base + cols[j]], acc[j],
                                   mask=cur_validm)
          return rcarry

        lax.fori_loop(0, _B // _NLANE, grp_body, 0)

        @pl.when(kb + 2 < blk_hi)
        def _next():
          start_fetch(kb + 2, buf)

      def pair_body(i, carry):
        for buf in range(2):
          kb = blk_lo + 2 * i + buf

          @pl.when(kb < blk_hi)
          def _consume():
            consume(buf, kb)

        return carry

      npairs = (blk_hi - blk_lo + 1) // 2
      lax.fori_loop(0, npairs, pair_body, 0)

      @pl.when(c == 0)
      def _zero_padding_row():
        z = jnp.zeros((_NLANE,), jnp.float32)
        for j in range(_D // _NLANE):
          accum_v[pl.ds(16 * j, _NLANE)] = z

      pltpu.sync_copy(accum_v, out_hbm.at[pl.ds(lo * _D, _CH * _D)])

  return scatter_add_kernel


_SC_KERNEL = _make_sc_kernel()


@jax.jit
def kernel(grad, sort_indices, pos_idx):
  del pos_idx
  g = grad.reshape(-1)
  idx = sort_indices.reshape(-1).astype(jnp.int32)
  bounds = jnp.arange(_NCHUNK + 1, dtype=jnp.int32) * _CH
  starts = jnp.searchsorted(idx, bounds, side="left").astype(jnp.int32)
  starts_padded = jnp.zeros((256,), jnp.int32).at[: _NCHUNK + 1].set(starts)
  zeros = jnp.zeros((_CH * _D,), jnp.float32)
  out = _SC_KERNEL(g, idx, starts_padded, zeros)
  return out.reshape(_VPAD, _D)[:_NUM_ROWS_OUT]


# speedup vs baseline: 6.2989x; 1.5861x over previous
import functools

import jax
import jax.numpy as jnp
from jax import lax
from jax.experimental import pallas as pl
from jax.experimental.pallas import tpu as pltpu
from jax.experimental.pallas import tpu_sc as plsc

_NUM_ROWS_OUT = 100000
_D = 64
_RS = 128
_N = 4096 * 200
_B = 128
_TPB = _B // 8
_NW = 32
_P = 6
_NCHUNK = _NW * _P
_CH = 528
_VPAD = _NCHUNK * _CH
_NLANE = 16


def _make_sc_kernel():
  mesh = plsc.VectorSubcoreMesh(core_axis_name="c", subcore_axis_name="s")

  @functools.partial(
      pl.kernel,
      mesh=mesh,
      compiler_params=pltpu.CompilerParams(needs_layout_passes=False),
      out_type=jax.ShapeDtypeStruct((_VPAD * _RS,), jnp.float32),
      scratch_types=[
          pltpu.VMEM((256,), jnp.int32),
          pltpu.VMEM((2, _B), jnp.int32),
          pltpu.VMEM((2, _TPB, 8, _D), jnp.float32),
          pltpu.VMEM((_CH * _RS,), jnp.float32),
          pltpu.SemaphoreType.DMA,
          pltpu.SemaphoreType.DMA,
          pltpu.SemaphoreType.DMA,
          pltpu.SemaphoreType.DMA,
      ],
  )
  def scatter_add_kernel(grad_hbm, idx_hbm, starts_hbm, zeros_hbm, out_hbm,
                         starts_v, idxblk_v, stage_v, accum_v,
                         gsem0, gsem1, isem0, isem1):
    wid = lax.axis_index("s") * 2 + lax.axis_index("c")
    pltpu.sync_copy(starts_hbm, starts_v)
    lane = lax.iota(jnp.int32, _NLANE)
    cols = [lane + 16 * j for j in range(_D // _NLANE)]
    gsems = (gsem0, gsem1)
    isems = (isem0, isem1)

    def start_fetch(k, buf):
      pltpu.make_async_copy(
          grad_hbm.at[pl.ds(k * _TPB, _TPB)],
          stage_v.at[buf], gsems[buf]).start()
      pltpu.make_async_copy(
          idx_hbm.at[pl.ds(k * _B, _B)],
          idxblk_v.at[buf], isems[buf]).start()

    def wait_fetch(buf):
      pltpu.make_async_copy(
          grad_hbm.at[pl.ds(0, _TPB)],
          stage_v.at[buf], gsems[buf]).wait()
      pltpu.make_async_copy(
          idx_hbm.at[pl.ds(0, _B)],
          idxblk_v.at[buf], isems[buf]).wait()

    for p in range(_P):
      c = wid * _P + p
      lo = c * _CH
      pltpu.sync_copy(zeros_hbm, accum_v)
      bounds_vec = starts_v[pl.ds(c, _NLANE)]
      row_start = bounds_vec[0]
      row_end = bounds_vec[1]
      blk_lo = row_start // _B
      blk_hi = (row_end + _B - 1) // _B

      @pl.when(blk_lo < blk_hi)
      def _prime0():
        start_fetch(blk_lo, 0)

      @pl.when(blk_lo + 1 < blk_hi)
      def _prime1():
        start_fetch(blk_lo + 1, 1)

      def consume(buf, kb):
        wait_fetch(buf)

        def grp_body(g, rcarry):
          locv16 = idxblk_v[buf, pl.ds(g * _NLANE, _NLANE)] - lo
          nj = _D // _NLANE
          cur_locb = jnp.full((_NLANE,), -(2 ** 30), dtype=jnp.int32)
          cur_base = cur_locb
          cur_validm = cur_locb < cur_locb
          acc = [jnp.zeros((_NLANE,), jnp.float32) for _ in range(nj)]
          for t in range(_NLANE):
            locb = jnp.full((_NLANE,), locv16[t], dtype=jnp.int32)
            samev = locb == cur_locb
            flushm = (~samev) & cur_validm
            v = [stage_v[buf, 2 * g + t // 8, t % 8, pl.ds(16 * j, _NLANE)]
                 for j in range(nj)]
            for j in range(nj):
              plsc.addupdate_scatter(accum_v, [cur_base + cols[j]], acc[j],
                                     mask=flushm)
            for j in range(nj):
              acc[j] = jnp.where(samev, acc[j] + v[j], v[j])
            cur_locb = locb
            cur_base = locb * _RS
            cur_validm = (locb >= 0) & (locb < _CH)
          for j in range(nj):
            plsc.addupdate_scatter(accum_v, [cur_base + cols[j]], acc[j],
                                   mask=cur_validm)
          return rcarry

        lax.fori_loop(0, _B // _NLANE, grp_body, 0)

        @pl.when(kb + 2 < blk_hi)
        def _next():
          start_fetch(kb + 2, buf)

      def pair_body(i, carry):
        for buf in range(2):
          kb = blk_lo + 2 * i + buf

          @pl.when(kb < blk_hi)
          def _consume():
            consume(buf, kb)

        return carry

      npairs = (blk_hi - blk_lo + 1) // 2
      lax.fori_loop(0, npairs, pair_body, 0)

      @pl.when(c == 0)
      def _zero_padding_row():
        z = jnp.zeros((_NLANE,), jnp.float32)
        for j in range(_D // _NLANE):
          accum_v[pl.ds(16 * j, _NLANE)] = z

      pltpu.sync_copy(accum_v, out_hbm.at[pl.ds(lo * _RS, _CH * _RS)])

  return scatter_add_kernel


_SC_KERNEL = _make_sc_kernel()


@jax.jit
def kernel(grad, sort_indices, pos_idx):
  del pos_idx
  g = grad.reshape(-1, 8, _D)
  idx = sort_indices.reshape(-1).astype(jnp.int32)
  bounds = jnp.arange(_NCHUNK + 1, dtype=jnp.int32) * _CH
  starts = jnp.searchsorted(idx, bounds, side="left").astype(jnp.int32)
  starts_padded = jnp.zeros((256,), jnp.int32).at[: _NCHUNK + 1].set(starts)
  zeros = jnp.zeros((_CH * _RS,), jnp.float32)
  out = _SC_KERNEL(g, idx, starts_padded, zeros)
  return out.reshape(_VPAD, _RS)[:_NUM_ROWS_OUT, :_D]
